# baseline (speedup 1.0000x reference)
import functools

import jax
import jax.numpy as jnp
from jax import lax
from jax.experimental import pallas as pl
from jax.experimental.pallas import tpu as pltpu
from jax.experimental.pallas import tpu_sc as plsc

NUM_CORES = 2
NUM_SUBCORES = 16
LANES = 16
NUM_WORKERS = NUM_CORES * NUM_SUBCORES
BATCH = 16384
DIM = 32
BPW = BATCH // NUM_WORKERS


def _mf_body(uidx_hbm, iidx_hbm, u_hbm, i_hbm, ubias_hbm, ibias_hbm, gbias_hbm,
             out_hbm,
             uidx_v, iidx_v, urows_v, irows_v, ub_v, ib_v, out_v, gb_v,
             sem):
    wid = lax.axis_index("s") * NUM_CORES + lax.axis_index("c")
    base = wid * BPW

    pltpu.sync_copy(uidx_hbm.at[pl.ds(base, BPW)], uidx_v)
    pltpu.sync_copy(iidx_hbm.at[pl.ds(base, BPW)], iidx_v)
    pltpu.sync_copy(gbias_hbm, gb_v)

    c0 = pltpu.async_copy(u_hbm.at[uidx_v], urows_v, sem)
    c1 = pltpu.async_copy(i_hbm.at[iidx_v], irows_v, sem)
    c2 = pltpu.async_copy(ubias_hbm.at[uidx_v], ub_v, sem)
    c3 = pltpu.async_copy(ibias_hbm.at[iidx_v], ib_v, sem)
    c0.wait()
    c1.wait()
    c2.wait()
    c3.wait()

    g = gb_v[...]
    def blk(b, carry):
        rids = b * LANES + lax.iota(jnp.int32, LANES)
        acc = ub_v[pl.ds(b * LANES, LANES)] + ib_v[pl.ds(b * LANES, LANES)] + g
        for d in range(DIM):
            dv = jnp.full((LANES,), d, jnp.int32)
            acc = acc + (plsc.load_gather(urows_v, [rids, dv]) *
                         plsc.load_gather(irows_v, [rids, dv]))
        out_v[pl.ds(b * LANES, LANES)] = acc
        return carry

    lax.fori_loop(0, BPW // LANES, blk, 0)

    pltpu.sync_copy(out_v, out_hbm.at[pl.ds(base, BPW)])


@functools.partial(jax.jit, donate_argnums=())
def kernel(u_idx, i_idx, U, I, user_bias, item_bias, global_bias):
    mesh = plsc.VectorSubcoreMesh(core_axis_name="c", subcore_axis_name="s",
                                  num_cores=NUM_CORES,
                                  num_subcores=NUM_SUBCORES)
    run = pl.kernel(
        _mf_body,
        out_type=jax.ShapeDtypeStruct((BATCH,), jnp.float32),
        mesh=mesh,
        scratch_types=[
            pltpu.VMEM((BPW,), jnp.int32),
            pltpu.VMEM((BPW,), jnp.int32),
            pltpu.VMEM((BPW, DIM), jnp.float32),
            pltpu.VMEM((BPW, DIM), jnp.float32),
            pltpu.VMEM((BPW,), jnp.float32),
            pltpu.VMEM((BPW,), jnp.float32),
            pltpu.VMEM((BPW,), jnp.float32),
            pltpu.VMEM((LANES,), jnp.float32),
            pltpu.SemaphoreType.DMA,
        ],
        compiler_params=pltpu.CompilerParams(needs_layout_passes=False, use_tc_tiling_on_sc=False),
    )
    gb = jnp.full((LANES,), global_bias, dtype=jnp.float32)
    return run(u_idx.astype(jnp.int32), i_idx.astype(jnp.int32),
               U, I, user_bias, item_bias, gb)
